# Initial kernel scaffold; baseline (speedup 1.0000x reference)
import jax
import jax.numpy as jnp
from jax.experimental import pallas as pl


def kernel(pc, sa0_w0, sa0_b0, sa0_w1, sa0_b1, sa1_w0, sa1_b0, sa1_w1, sa1_b1, sa2_w0, sa2_b0, sa2_w1, sa2_b1, sa3_w0, sa3_b0, sa3_w1, sa3_b1, fc1_w, fc1_b, fc2_w, fc2_b, bn_gamma, bn_beta):
    raise NotImplementedError("write your pallas kernel here")


# speedup vs baseline: 11.2880x; 11.2880x over previous
import functools

import jax
import jax.numpy as jnp
from jax.experimental import pallas as pl

_NEG = -1.7e38


def _fps_kernel(xyz_ref, out_ref, *, npoint):
    xyz = xyz_ref[...]
    _, B, N = xyz.shape
    iota = jax.lax.broadcasted_iota(jnp.int32, (B, N), 1)
    x0, x1, x2 = xyz[0], xyz[1], xyz[2]

    def step(t, carry):
        dists, far = carry
        sel = iota == far
        c0 = jnp.sum(jnp.where(sel, x0, 0.0), axis=-1, keepdims=True)
        c1 = jnp.sum(jnp.where(sel, x1, 0.0), axis=-1, keepdims=True)
        c2 = jnp.sum(jnp.where(sel, x2, 0.0), axis=-1, keepdims=True)
        out_ref[pl.ds(t, 1)] = jnp.concatenate([c0, c1, c2], axis=-1)[None]
        d0 = x0 - c0
        d1 = x1 - c1
        d2 = x2 - c2
        d = d0 * d0 + d1 * d1 + d2 * d2
        dists = jnp.minimum(dists, d)
        m = jnp.max(dists, axis=-1, keepdims=True)
        far = jnp.min(jnp.where(dists == m, iota, N), axis=-1, keepdims=True)
        return dists, far

    d0 = jnp.full((B, N), 1e10, dtype=jnp.float32)
    f0 = jnp.zeros((B, 1), dtype=jnp.int32)
    jax.lax.fori_loop(0, npoint, step, (d0, f0))


def _fps(xyz3, npoint):
    _, B, N = xyz3.shape
    return pl.pallas_call(
        functools.partial(_fps_kernel, npoint=npoint),
        out_shape=jax.ShapeDtypeStruct((npoint, B, 3), jnp.float32),
    )(xyz3)


def _mlp_kernel(x_ref, w0_ref, b0_ref, w1_ref, b1_ref, out_ref):
    x = x_ref[0]
    h = jnp.maximum(
        jnp.dot(x, w0_ref[...], preferred_element_type=jnp.float32)
        + b0_ref[...], 0.0)
    out_ref[0] = jnp.maximum(
        jnp.dot(h, w1_ref[...], preferred_element_type=jnp.float32)
        + b1_ref[...], 0.0)


def _mlp(x, w0t, b0, w1t, b1):
    B, N, Cin = x.shape
    C1 = w0t.shape[1]
    C2 = w1t.shape[1]
    return pl.pallas_call(
        _mlp_kernel,
        grid=(B,),
        in_specs=[
            pl.BlockSpec((1, N, Cin), lambda b: (b, 0, 0)),
            pl.BlockSpec((Cin, C1), lambda b: (0, 0)),
            pl.BlockSpec((1, C1), lambda b: (0, 0)),
            pl.BlockSpec((C1, C2), lambda b: (0, 0)),
            pl.BlockSpec((1, C2), lambda b: (0, 0)),
        ],
        out_specs=pl.BlockSpec((1, N, C2), lambda b: (b, 0, 0)),
        out_shape=jax.ShapeDtypeStruct((B, N, C2), jnp.float32),
    )(x, w0t, b0, w1t, b1)


def _maskmax_kernel(xyz_ref, nxT_ref, v_ref, out_ref, *, r2, nsample, npoint):
    xyz = xyz_ref[0]
    nxT = nxT_ref[0]
    v = v_ref[0]
    N = xyz.shape[0]

    x2 = jnp.sum(xyz * xyz, axis=-1, keepdims=True)
    cn2 = jnp.sum(nxT * nxT, axis=0, keepdims=True)
    dotT = jnp.dot(xyz, nxT, preferred_element_type=jnp.float32)
    sqT = x2 + cn2 - 2.0 * dotT
    mask = jnp.where(sqT <= r2, 1.0, 0.0)

    csum = mask
    sh = 1
    while sh < N:
        shifted = jnp.concatenate(
            [jnp.zeros((sh, csum.shape[1]), jnp.float32), csum[:-sh]], axis=0)
        csum = csum + shifted
        sh *= 2

    msel = jnp.where((mask > 0.0) & (csum <= nsample), 0.0, _NEG)
    sio = jax.lax.broadcasted_iota(jnp.int32, (npoint, 1), 0)

    def body(s, _):
        onehot = jnp.where(sio == s, 1.0, 0.0)
        col = jnp.dot(msel, onehot, preferred_element_type=jnp.float32,
                      precision=jax.lax.Precision.HIGHEST)
        red = jnp.max(v + col, axis=0)
        out_ref[0, pl.ds(s, 1)] = red[None]
        return 0

    jax.lax.fori_loop(0, npoint, body, 0)


def _maskmax(xyz, new_xyzT, v, radius, nsample):
    B, N, _ = xyz.shape
    S = new_xyzT.shape[2]
    C = v.shape[2]
    return pl.pallas_call(
        functools.partial(_maskmax_kernel, r2=radius * radius,
                          nsample=nsample, npoint=S),
        grid=(B,),
        in_specs=[
            pl.BlockSpec((1, N, 3), lambda b: (b, 0, 0)),
            pl.BlockSpec((1, 3, S), lambda b: (b, 0, 0)),
            pl.BlockSpec((1, N, C), lambda b: (b, 0, 0)),
        ],
        out_specs=pl.BlockSpec((1, S, C), lambda b: (b, 0, 0)),
        out_shape=jax.ShapeDtypeStruct((B, S, C), jnp.float32),
    )(xyz, new_xyzT, v)


def _tail_kernel(x_ref, w0_ref, b0_ref, w1_ref, b1_ref, fc1_ref, fb1_ref,
                 fc2_ref, fb2_ref, g_ref, bt_ref, out_ref):
    B, N3, C = x_ref.shape
    x = x_ref[...].reshape(B * N3, C)
    h = jnp.maximum(
        jnp.dot(x, w0_ref[...], preferred_element_type=jnp.float32)
        + b0_ref[...], 0.0)
    h = jnp.maximum(
        jnp.dot(h, w1_ref[...], preferred_element_type=jnp.float32)
        + b1_ref[...], 0.0)
    f = jnp.max(h.reshape(B, N3, h.shape[-1]), axis=1)
    y = jnp.maximum(
        jnp.dot(f, fc1_ref[...], preferred_element_type=jnp.float32)
        + fb1_ref[...], 0.0)
    y = jnp.dot(y, fc2_ref[...], preferred_element_type=jnp.float32) \
        + fb2_ref[...]
    scale = jnp.float32(1.0) / jnp.sqrt(jnp.float32(1.0 + 1e-5))
    out_ref[...] = (y * scale) * g_ref[...] + bt_ref[...]


def _tail(x, w0t, b0, w1t, b1, fc1t, fb1, fc2t, fb2, g, bt):
    B = x.shape[0]
    return pl.pallas_call(
        _tail_kernel,
        out_shape=jax.ShapeDtypeStruct((B, 128), jnp.float32),
    )(x, w0t, b0, w1t, b1, fc1t, fb1, fc2t, fb2, g, bt)


_NPOINTS = (256, 64, 16)
_RADII = (0.1, 0.2, 0.4)
_NSAMPLES = (64, 64, 32)


def kernel(pc, sa0_w0, sa0_b0, sa0_w1, sa0_b1, sa1_w0, sa1_b0, sa1_w1,
           sa1_b1, sa2_w0, sa2_b0, sa2_w1, sa2_b1, sa3_w0, sa3_b0, sa3_w1,
           sa3_b1, fc1_w, fc1_b, fc2_w, fc2_b, bn_gamma, bn_beta):
    B, N, _ = pc.shape
    xyz = pc[..., :3]
    feats = jnp.concatenate([pc[..., 3:], xyz], axis=-1)

    ws = [(sa0_w0, sa0_b0, sa0_w1, sa0_b1),
          (sa1_w0, sa1_b0, sa1_w1, sa1_b1),
          (sa2_w0, sa2_b0, sa2_w1, sa2_b1)]

    xyz3 = jnp.transpose(xyz, (2, 0, 1))
    for k in range(3):
        w0, b0, w1, b1 = ws[k]
        cen = _fps(xyz3, _NPOINTS[k])
        v = _mlp(feats, w0.T, b0[None, :], w1.T, b1[None, :])
        new_xyzT = jnp.transpose(cen, (1, 2, 0))
        feats = _maskmax(xyz, new_xyzT, v, _RADII[k], _NSAMPLES[k])
        xyz = jnp.transpose(cen, (1, 0, 2))
        xyz3 = jnp.transpose(cen, (2, 1, 0))

    return _tail(feats, sa3_w0.T, sa3_b0[None, :], sa3_w1.T, sa3_b1[None, :],
                 fc1_w.T, fc1_b[None, :], fc2_w.T, fc2_b[None, :],
                 bn_gamma[None, :], bn_beta[None, :])


# speedup vs baseline: 93.9610x; 8.3240x over previous
import functools

import jax
import jax.numpy as jnp
from jax.experimental import pallas as pl
from jax.experimental.pallas import tpu as pltpu

_NEG = -1.7e38


def _fps_kernel(xyz_ref, out_ref, *, npoint):
    xyz = xyz_ref[...]
    _, B, N = xyz.shape
    iota = jax.lax.broadcasted_iota(jnp.int32, (B, N), 1)
    x0, x1, x2 = xyz[0], xyz[1], xyz[2]

    def step(t, carry):
        dists, far = carry
        sel = iota == far
        c0 = jnp.sum(jnp.where(sel, x0, 0.0), axis=-1, keepdims=True)
        c1 = jnp.sum(jnp.where(sel, x1, 0.0), axis=-1, keepdims=True)
        c2 = jnp.sum(jnp.where(sel, x2, 0.0), axis=-1, keepdims=True)
        out_ref[pl.ds(t, 1)] = jnp.concatenate([c0, c1, c2], axis=-1)[None]
        d0 = x0 - c0
        d1 = x1 - c1
        d2 = x2 - c2
        d = d0 * d0 + d1 * d1 + d2 * d2
        dists = jnp.minimum(dists, d)
        m = jnp.max(dists, axis=-1, keepdims=True)
        far = jnp.min(jnp.where(dists == m, iota, N), axis=-1, keepdims=True)
        return dists, far

    d0 = jnp.full((B, N), 1e10, dtype=jnp.float32)
    f0 = jnp.zeros((B, 1), dtype=jnp.int32)
    jax.lax.fori_loop(0, npoint, step, (d0, f0))


def _fps(xyz3, npoint):
    _, B, N = xyz3.shape
    return pl.pallas_call(
        functools.partial(_fps_kernel, npoint=npoint),
        out_shape=jax.ShapeDtypeStruct((npoint, B, 3), jnp.float32),
    )(xyz3)


def _mlp_kernel(x_ref, w0_ref, b0_ref, w1_ref, b1_ref, out_ref):
    x = x_ref[0]
    h = jnp.maximum(
        jnp.dot(x, w0_ref[...], preferred_element_type=jnp.float32)
        + b0_ref[...], 0.0)
    out_ref[0] = jnp.maximum(
        jnp.dot(h, w1_ref[...], preferred_element_type=jnp.float32)
        + b1_ref[...], 0.0)


def _mlp(x, w0t, b0, w1t, b1):
    B, N, Cin = x.shape
    C1 = w0t.shape[1]
    C2 = w1t.shape[1]
    return pl.pallas_call(
        _mlp_kernel,
        grid=(B,),
        in_specs=[
            pl.BlockSpec((1, N, Cin), lambda b: (b, 0, 0)),
            pl.BlockSpec((Cin, C1), lambda b: (0, 0)),
            pl.BlockSpec((1, C1), lambda b: (0, 0)),
            pl.BlockSpec((C1, C2), lambda b: (0, 0)),
            pl.BlockSpec((1, C2), lambda b: (0, 0)),
        ],
        out_specs=pl.BlockSpec((1, N, C2), lambda b: (b, 0, 0)),
        out_shape=jax.ShapeDtypeStruct((B, N, C2), jnp.float32),
    )(x, w0t, b0, w1t, b1)


def _maskmax_kernel(nx_ref, xyzT_ref, v_ref, out_ref, msel_ref,
                    *, r2, nsample, npoint):
    nx = nx_ref[0]
    xyzT = xyzT_ref[0]
    v = v_ref[0]
    N = xyzT.shape[1]

    cn2 = jnp.sum(nx * nx, axis=-1, keepdims=True)
    x2 = jnp.sum(xyzT * xyzT, axis=0, keepdims=True)
    dt = jnp.dot(nx, xyzT, preferred_element_type=jnp.float32)
    sq = cn2 + x2 - 2.0 * dt
    mask = jnp.where(sq <= r2, 1.0, 0.0)

    csum = mask
    sh = 1
    while sh < N:
        shifted = jnp.concatenate(
            [jnp.zeros((csum.shape[0], sh), jnp.float32), csum[:, :-sh]],
            axis=1)
        csum = csum + shifted
        sh *= 2

    msel_ref[...] = jnp.where((mask > 0.0) & (csum <= nsample), 0.0, _NEG)

    def body(s, _):
        row = msel_ref[pl.ds(s, 1), :]
        red = jnp.max(v + row, axis=1)
        out_ref[0, pl.ds(s, 1)] = red[None]
        return 0

    jax.lax.fori_loop(0, npoint, body, 0)


def _maskmax(nx, xyzT, vT, radius, nsample):
    B, S, _ = nx.shape
    N = xyzT.shape[2]
    C = vT.shape[1]
    return pl.pallas_call(
        functools.partial(_maskmax_kernel, r2=radius * radius,
                          nsample=nsample, npoint=S),
        grid=(B,),
        in_specs=[
            pl.BlockSpec((1, S, 3), lambda b: (b, 0, 0)),
            pl.BlockSpec((1, 3, N), lambda b: (b, 0, 0)),
            pl.BlockSpec((1, C, N), lambda b: (b, 0, 0)),
        ],
        out_specs=pl.BlockSpec((1, S, C), lambda b: (b, 0, 0)),
        out_shape=jax.ShapeDtypeStruct((B, S, C), jnp.float32),
        scratch_shapes=[pltpu.VMEM((S, N), jnp.float32)],
    )(nx, xyzT, vT)


def _tail_kernel(x_ref, w0_ref, b0_ref, w1_ref, b1_ref, fc1_ref, fb1_ref,
                 fc2_ref, fb2_ref, g_ref, bt_ref, out_ref):
    B, N3, C = x_ref.shape
    x = x_ref[...].reshape(B * N3, C)
    h = jnp.maximum(
        jnp.dot(x, w0_ref[...], preferred_element_type=jnp.float32)
        + b0_ref[...], 0.0)
    h = jnp.maximum(
        jnp.dot(h, w1_ref[...], preferred_element_type=jnp.float32)
        + b1_ref[...], 0.0)
    f = jnp.max(h.reshape(B, N3, h.shape[-1]), axis=1)
    y = jnp.maximum(
        jnp.dot(f, fc1_ref[...], preferred_element_type=jnp.float32)
        + fb1_ref[...], 0.0)
    y = jnp.dot(y, fc2_ref[...], preferred_element_type=jnp.float32) \
        + fb2_ref[...]
    scale = jnp.float32(1.0) / jnp.sqrt(jnp.float32(1.0 + 1e-5))
    out_ref[...] = (y * scale) * g_ref[...] + bt_ref[...]


def _tail(x, w0t, b0, w1t, b1, fc1t, fb1, fc2t, fb2, g, bt):
    B = x.shape[0]
    return pl.pallas_call(
        _tail_kernel,
        out_shape=jax.ShapeDtypeStruct((B, 128), jnp.float32),
    )(x, w0t, b0, w1t, b1, fc1t, fb1, fc2t, fb2, g, bt)


_NPOINTS = (256, 64, 16)
_RADII = (0.1, 0.2, 0.4)
_NSAMPLES = (64, 64, 32)


def kernel(pc, sa0_w0, sa0_b0, sa0_w1, sa0_b1, sa1_w0, sa1_b0, sa1_w1,
           sa1_b1, sa2_w0, sa2_b0, sa2_w1, sa2_b1, sa3_w0, sa3_b0, sa3_w1,
           sa3_b1, fc1_w, fc1_b, fc2_w, fc2_b, bn_gamma, bn_beta):
    B, N, _ = pc.shape
    xyz = pc[..., :3]
    feats = jnp.concatenate([pc[..., 3:], xyz], axis=-1)

    ws = [(sa0_w0, sa0_b0, sa0_w1, sa0_b1),
          (sa1_w0, sa1_b0, sa1_w1, sa1_b1),
          (sa2_w0, sa2_b0, sa2_w1, sa2_b1)]

    xyz3 = jnp.transpose(xyz, (2, 0, 1))
    xyzT = jnp.transpose(xyz, (0, 2, 1))
    for k in range(3):
        w0, b0, w1, b1 = ws[k]
        cen = _fps(xyz3, _NPOINTS[k])
        v = _mlp(feats, w0.T, b0[None, :], w1.T, b1[None, :])
        vT = jnp.transpose(v, (0, 2, 1))
        nx = jnp.transpose(cen, (1, 0, 2))
        feats = _maskmax(nx, xyzT, vT, _RADII[k], _NSAMPLES[k])
        xyzT = jnp.transpose(cen, (1, 2, 0))
        xyz3 = jnp.transpose(cen, (2, 1, 0))

    return _tail(feats, sa3_w0.T, sa3_b0[None, :], sa3_w1.T, sa3_b1[None, :],
                 fc1_w.T, fc1_b[None, :], fc2_w.T, fc2_b[None, :],
                 bn_gamma[None, :], bn_beta[None, :])


# speedup vs baseline: 130.1786x; 1.3855x over previous
import functools

import jax
import jax.numpy as jnp
from jax.experimental import pallas as pl
from jax.experimental.pallas import tpu as pltpu

_NEG = -1.7e38


def _fps_kernel(xyz_ref, out_ref, *, npoint):
    xyz = xyz_ref[...]
    _, B, N = xyz.shape
    iota = jax.lax.broadcasted_iota(jnp.int32, (B, N), 1)
    x0, x1, x2 = xyz[0], xyz[1], xyz[2]

    def step(t, carry):
        dists, far = carry
        sel = iota == far
        c0 = jnp.sum(jnp.where(sel, x0, 0.0), axis=-1, keepdims=True)
        c1 = jnp.sum(jnp.where(sel, x1, 0.0), axis=-1, keepdims=True)
        c2 = jnp.sum(jnp.where(sel, x2, 0.0), axis=-1, keepdims=True)
        out_ref[pl.ds(t, 1)] = jnp.concatenate([c0, c1, c2], axis=-1)[None]
        d0 = x0 - c0
        d1 = x1 - c1
        d2 = x2 - c2
        d = d0 * d0 + d1 * d1 + d2 * d2
        dists = jnp.minimum(dists, d)
        m = jnp.max(dists, axis=-1, keepdims=True)
        far = jnp.min(jnp.where(dists == m, iota, N), axis=-1, keepdims=True)
        return dists, far

    d0 = jnp.full((B, N), 1e10, dtype=jnp.float32)
    f0 = jnp.zeros((B, 1), dtype=jnp.int32)
    jax.lax.fori_loop(0, npoint, step, (d0, f0))


def _fps(xyz3, npoint):
    _, B, N = xyz3.shape
    return pl.pallas_call(
        functools.partial(_fps_kernel, npoint=npoint),
        out_shape=jax.ShapeDtypeStruct((npoint, B, 3), jnp.float32),
    )(xyz3)


def _mlp_kernel(x_ref, w0_ref, b0_ref, w1_ref, b1_ref, out_ref):
    x = x_ref[0]
    h = jnp.maximum(
        jnp.dot(x, w0_ref[...], preferred_element_type=jnp.float32)
        + b0_ref[...], 0.0)
    out_ref[0] = jnp.maximum(
        jnp.dot(h, w1_ref[...], preferred_element_type=jnp.float32)
        + b1_ref[...], 0.0)


def _mlp(x, w0t, b0, w1t, b1):
    B, N, Cin = x.shape
    C1 = w0t.shape[1]
    C2 = w1t.shape[1]
    return pl.pallas_call(
        _mlp_kernel,
        grid=(B,),
        in_specs=[
            pl.BlockSpec((1, N, Cin), lambda b: (b, 0, 0)),
            pl.BlockSpec((Cin, C1), lambda b: (0, 0)),
            pl.BlockSpec((1, C1), lambda b: (0, 0)),
            pl.BlockSpec((C1, C2), lambda b: (0, 0)),
            pl.BlockSpec((1, C2), lambda b: (0, 0)),
        ],
        out_specs=pl.BlockSpec((1, N, C2), lambda b: (b, 0, 0)),
        out_shape=jax.ShapeDtypeStruct((B, N, C2), jnp.float32),
    )(x, w0t, b0, w1t, b1)


def _maskmax_kernel(nx_ref, xyzT_ref, v_ref, out_ref, msel_ref,
                    *, r2, nsample, npoint):
    nx = nx_ref[0]
    xyzT = xyzT_ref[0]
    v = v_ref[0]
    N = xyzT.shape[1]

    cn2 = jnp.sum(nx * nx, axis=-1, keepdims=True)
    x2 = jnp.sum(xyzT * xyzT, axis=0, keepdims=True)
    dt = jnp.dot(nx, xyzT, preferred_element_type=jnp.float32)
    sq = cn2 + x2 - 2.0 * dt
    mask = jnp.where(sq <= r2, 1.0, 0.0)

    csum = mask
    sh = 1
    while sh < N:
        shifted = jnp.concatenate(
            [jnp.zeros((csum.shape[0], sh), jnp.float32), csum[:, :-sh]],
            axis=1)
        csum = csum + shifted
        sh *= 2

    msel_ref[...] = jnp.where((mask > 0.0) & (csum <= nsample), 0.0, _NEG)

    U = 8 if npoint % 8 == 0 else npoint

    def body(s0, _):
        rows = msel_ref[pl.ds(s0 * U, U), :]
        reds = [jnp.max(v + rows[j:j + 1, :], axis=1) for j in range(U)]
        out_ref[0, pl.ds(s0 * U, U)] = jnp.stack(reds, axis=0)
        return 0

    jax.lax.fori_loop(0, npoint // U, body, 0)


def _maskmax(nx, xyzT, vT, radius, nsample):
    B, S, _ = nx.shape
    N = xyzT.shape[2]
    C = vT.shape[1]
    return pl.pallas_call(
        functools.partial(_maskmax_kernel, r2=radius * radius,
                          nsample=nsample, npoint=S),
        grid=(B,),
        in_specs=[
            pl.BlockSpec((1, S, 3), lambda b: (b, 0, 0)),
            pl.BlockSpec((1, 3, N), lambda b: (b, 0, 0)),
            pl.BlockSpec((1, C, N), lambda b: (b, 0, 0)),
        ],
        out_specs=pl.BlockSpec((1, S, C), lambda b: (b, 0, 0)),
        out_shape=jax.ShapeDtypeStruct((B, S, C), jnp.float32),
        scratch_shapes=[pltpu.VMEM((S, N), jnp.float32)],
    )(nx, xyzT, vT)


def _tail_kernel(x_ref, w0_ref, b0_ref, w1_ref, b1_ref, fc1_ref, fb1_ref,
                 fc2_ref, fb2_ref, g_ref, bt_ref, out_ref):
    B, N3, C = x_ref.shape
    x = x_ref[...].reshape(B * N3, C)
    h = jnp.maximum(
        jnp.dot(x, w0_ref[...], preferred_element_type=jnp.float32)
        + b0_ref[...], 0.0)
    h = jnp.maximum(
        jnp.dot(h, w1_ref[...], preferred_element_type=jnp.float32)
        + b1_ref[...], 0.0)
    f = jnp.max(h.reshape(B, N3, h.shape[-1]), axis=1)
    y = jnp.maximum(
        jnp.dot(f, fc1_ref[...], preferred_element_type=jnp.float32)
        + fb1_ref[...], 0.0)
    y = jnp.dot(y, fc2_ref[...], preferred_element_type=jnp.float32) \
        + fb2_ref[...]
    scale = jnp.float32(1.0) / jnp.sqrt(jnp.float32(1.0 + 1e-5))
    out_ref[...] = (y * scale) * g_ref[...] + bt_ref[...]


def _tail(x, w0t, b0, w1t, b1, fc1t, fb1, fc2t, fb2, g, bt):
    B = x.shape[0]
    return pl.pallas_call(
        _tail_kernel,
        out_shape=jax.ShapeDtypeStruct((B, 128), jnp.float32),
    )(x, w0t, b0, w1t, b1, fc1t, fb1, fc2t, fb2, g, bt)


_NPOINTS = (256, 64, 16)
_RADII = (0.1, 0.2, 0.4)
_NSAMPLES = (64, 64, 32)


def kernel(pc, sa0_w0, sa0_b0, sa0_w1, sa0_b1, sa1_w0, sa1_b0, sa1_w1,
           sa1_b1, sa2_w0, sa2_b0, sa2_w1, sa2_b1, sa3_w0, sa3_b0, sa3_w1,
           sa3_b1, fc1_w, fc1_b, fc2_w, fc2_b, bn_gamma, bn_beta):
    B, N, _ = pc.shape
    xyz = pc[..., :3]
    feats = jnp.concatenate([pc[..., 3:], xyz], axis=-1)

    ws = [(sa0_w0, sa0_b0, sa0_w1, sa0_b1),
          (sa1_w0, sa1_b0, sa1_w1, sa1_b1),
          (sa2_w0, sa2_b0, sa2_w1, sa2_b1)]

    xyz3 = jnp.transpose(xyz, (2, 0, 1))
    xyzT = jnp.transpose(xyz, (0, 2, 1))
    for k in range(3):
        w0, b0, w1, b1 = ws[k]
        cen = _fps(xyz3, _NPOINTS[k])
        v = _mlp(feats, w0.T, b0[None, :], w1.T, b1[None, :])
        vT = jnp.transpose(v, (0, 2, 1))
        nx = jnp.transpose(cen, (1, 0, 2))
        feats = _maskmax(nx, xyzT, vT, _RADII[k], _NSAMPLES[k])
        xyzT = jnp.transpose(cen, (1, 2, 0))
        xyz3 = jnp.transpose(cen, (2, 1, 0))

    return _tail(feats, sa3_w0.T, sa3_b0[None, :], sa3_w1.T, sa3_b1[None, :],
                 fc1_w.T, fc1_b[None, :], fc2_w.T, fc2_b[None, :],
                 bn_gamma[None, :], bn_beta[None, :])


# speedup vs baseline: 139.7940x; 1.0739x over previous
import functools

import jax
import jax.numpy as jnp
from jax.experimental import pallas as pl
from jax.experimental.pallas import tpu as pltpu

_NEG = -1.7e38


def _fps_kernel(xyz_ref, out_ref, *, npoint):
    xyz = xyz_ref[...]
    _, B, N = xyz.shape
    iota = jax.lax.broadcasted_iota(jnp.int32, (B, N), 1)
    x0, x1, x2 = xyz[0], xyz[1], xyz[2]

    def step(t, carry):
        dists, far = carry
        sel = iota == far
        c0 = jnp.sum(jnp.where(sel, x0, 0.0), axis=-1, keepdims=True)
        c1 = jnp.sum(jnp.where(sel, x1, 0.0), axis=-1, keepdims=True)
        c2 = jnp.sum(jnp.where(sel, x2, 0.0), axis=-1, keepdims=True)
        out_ref[pl.ds(t, 1)] = jnp.concatenate([c0, c1, c2], axis=-1)[None]
        d0 = x0 - c0
        d1 = x1 - c1
        d2 = x2 - c2
        d = d0 * d0 + d1 * d1 + d2 * d2
        dists = jnp.minimum(dists, d)
        m = jnp.max(dists, axis=-1, keepdims=True)
        far = jnp.min(jnp.where(dists == m, iota, N), axis=-1, keepdims=True)
        return dists, far

    d0 = jnp.full((B, N), 1e10, dtype=jnp.float32)
    f0 = jnp.zeros((B, 1), dtype=jnp.int32)
    jax.lax.fori_loop(0, npoint, step, (d0, f0))


def _fps(xyz3, npoint):
    _, B, N = xyz3.shape
    return pl.pallas_call(
        functools.partial(_fps_kernel, npoint=npoint),
        out_shape=jax.ShapeDtypeStruct((npoint, B, 3), jnp.float32),
    )(xyz3)


def _maskmax_kernel(x_ref, w0_ref, b0_ref, w1_ref, b1_ref, nx_ref, xyzT_ref,
                    out_ref, msel_ref, *, r2, nsample, npoint):
    nx = nx_ref[0]
    xyzT = xyzT_ref[0]
    N = xyzT.shape[1]

    h = jnp.maximum(
        jnp.dot(w0_ref[...], x_ref[0], preferred_element_type=jnp.float32)
        + b0_ref[...], 0.0)
    v = jnp.maximum(
        jnp.dot(w1_ref[...], h, preferred_element_type=jnp.float32)
        + b1_ref[...], 0.0)

    cn2 = jnp.sum(nx * nx, axis=-1, keepdims=True)
    x2 = jnp.sum(xyzT * xyzT, axis=0, keepdims=True)
    dt = jnp.dot(nx, xyzT, preferred_element_type=jnp.float32)
    sq = cn2 + x2 - 2.0 * dt
    mask = jnp.where(sq <= r2, 1.0, 0.0)

    csum = mask
    sh = 1
    while sh < N:
        shifted = jnp.concatenate(
            [jnp.zeros((csum.shape[0], sh), jnp.float32), csum[:, :-sh]],
            axis=1)
        csum = csum + shifted
        sh *= 2

    msel_ref[...] = jnp.where((mask > 0.0) & (csum <= nsample), 0.0, _NEG)

    U = 8 if npoint % 8 == 0 else npoint

    def body(s0, _):
        rows = msel_ref[pl.ds(s0 * U, U), :]
        reds = [jnp.max(v + rows[j:j + 1, :], axis=1) for j in range(U)]
        out_ref[0, pl.ds(s0 * U, U)] = jnp.stack(reds, axis=0)
        return 0

    jax.lax.fori_loop(0, npoint // U, body, 0)


def _maskmax(x_cm, w0, b0, w1, b1, nx, xyzT, radius, nsample):
    B, S, _ = nx.shape
    N = xyzT.shape[2]
    Cin = x_cm.shape[1]
    C1 = w0.shape[0]
    C = w1.shape[0]
    return pl.pallas_call(
        functools.partial(_maskmax_kernel, r2=radius * radius,
                          nsample=nsample, npoint=S),
        grid=(B,),
        in_specs=[
            pl.BlockSpec((1, Cin, N), lambda b: (b, 0, 0)),
            pl.BlockSpec((C1, Cin), lambda b: (0, 0)),
            pl.BlockSpec((C1, 1), lambda b: (0, 0)),
            pl.BlockSpec((C, C1), lambda b: (0, 0)),
            pl.BlockSpec((C, 1), lambda b: (0, 0)),
            pl.BlockSpec((1, S, 3), lambda b: (b, 0, 0)),
            pl.BlockSpec((1, 3, N), lambda b: (b, 0, 0)),
        ],
        out_specs=pl.BlockSpec((1, S, C), lambda b: (b, 0, 0)),
        out_shape=jax.ShapeDtypeStruct((B, S, C), jnp.float32),
        scratch_shapes=[pltpu.VMEM((S, N), jnp.float32)],
    )(x_cm, w0, b0[:, None], w1, b1[:, None], nx, xyzT)


def _tail_kernel(x_ref, w0_ref, b0_ref, w1_ref, b1_ref, fc1_ref, fb1_ref,
                 fc2_ref, fb2_ref, g_ref, bt_ref, out_ref):
    B, N3, C = x_ref.shape
    x = x_ref[...].reshape(B * N3, C)
    h = jnp.maximum(
        jnp.dot(x, w0_ref[...], preferred_element_type=jnp.float32)
        + b0_ref[...], 0.0)
    h = jnp.maximum(
        jnp.dot(h, w1_ref[...], preferred_element_type=jnp.float32)
        + b1_ref[...], 0.0)
    f = jnp.max(h.reshape(B, N3, h.shape[-1]), axis=1)
    y = jnp.maximum(
        jnp.dot(f, fc1_ref[...], preferred_element_type=jnp.float32)
        + fb1_ref[...], 0.0)
    y = jnp.dot(y, fc2_ref[...], preferred_element_type=jnp.float32) \
        + fb2_ref[...]
    scale = jnp.float32(1.0) / jnp.sqrt(jnp.float32(1.0 + 1e-5))
    out_ref[...] = (y * scale) * g_ref[...] + bt_ref[...]


def _tail(x, w0t, b0, w1t, b1, fc1t, fb1, fc2t, fb2, g, bt):
    B = x.shape[0]
    return pl.pallas_call(
        _tail_kernel,
        out_shape=jax.ShapeDtypeStruct((B, 128), jnp.float32),
    )(x, w0t, b0, w1t, b1, fc1t, fb1, fc2t, fb2, g, bt)


_NPOINTS = (256, 64, 16)
_RADII = (0.1, 0.2, 0.4)
_NSAMPLES = (64, 64, 32)


def kernel(pc, sa0_w0, sa0_b0, sa0_w1, sa0_b1, sa1_w0, sa1_b0, sa1_w1,
           sa1_b1, sa2_w0, sa2_b0, sa2_w1, sa2_b1, sa3_w0, sa3_b0, sa3_w1,
           sa3_b1, fc1_w, fc1_b, fc2_w, fc2_b, bn_gamma, bn_beta):
    B, N, _ = pc.shape
    xyz = pc[..., :3]
    feats_cm = jnp.concatenate(
        [jnp.transpose(pc[..., 3:], (0, 2, 1)),
         jnp.transpose(xyz, (0, 2, 1))], axis=1)

    ws = [(sa0_w0, sa0_b0, sa0_w1, sa0_b1),
          (sa1_w0, sa1_b0, sa1_w1, sa1_b1),
          (sa2_w0, sa2_b0, sa2_w1, sa2_b1)]

    xyz3 = jnp.transpose(xyz, (2, 0, 1))
    xyzT = jnp.transpose(xyz, (0, 2, 1))
    for k in range(3):
        w0, b0, w1, b1 = ws[k]
        cen = _fps(xyz3, _NPOINTS[k])
        nx = jnp.transpose(cen, (1, 0, 2))
        feats = _maskmax(feats_cm, w0, b0, w1, b1, nx, xyzT,
                         _RADII[k], _NSAMPLES[k])
        feats_cm = jnp.transpose(feats, (0, 2, 1))
        xyzT = jnp.transpose(cen, (1, 2, 0))
        xyz3 = jnp.transpose(cen, (2, 1, 0))

    return _tail(feats, sa3_w0.T, sa3_b0[None, :], sa3_w1.T, sa3_b1[None, :],
                 fc1_w.T, fc1_b[None, :], fc2_w.T, fc2_b[None, :],
                 bn_gamma[None, :], bn_beta[None, :])
